# initial kernel scaffold (unmeasured)
import jax
import jax.numpy as jnp
from jax import lax
from jax.experimental import pallas as pl
from jax.experimental.pallas import tpu as pltpu

N_DEV = 4


def kernel(x, w_mat):
    m_per, k = x.shape
    _, n_per = w_mat.shape
    half = m_per // 2

    def body(x_ref, w_ref, out_ref, commT, commB,
             copy_sems, send_semsT, recv_semsT, send_semsB, recv_semsB):
        my = lax.axis_index("i")
        left = (my - 1 + N_DEV) % N_DEV
        right = (my + 1) % N_DEV

        cpT = pltpu.make_async_copy(
            x_ref.at[pl.ds(0, half)], commT.at[0], copy_sems.at[0])
        cpB = pltpu.make_async_copy(
            x_ref.at[pl.ds(half, half)], commB.at[0], copy_sems.at[1])
        cpT.start()
        cpB.start()

        barrier = pltpu.get_barrier_semaphore()
        for nbr in (left, right):
            pl.semaphore_signal(
                barrier, inc=1,
                device_id=(nbr,), device_id_type=pl.DeviceIdType.MESH)
        pl.semaphore_wait(barrier, 2)
        cpT.wait()
        cpB.wait()

        def gemm(buf, slot, origin, row_off):
            acc = jnp.dot(buf[slot], w_ref[...],
                          preferred_element_type=jnp.float32)
            out_ref[pl.ds(origin * m_per + row_off, half), :] = acc

        for h in range(N_DEV - 1):
            s, r = h % 2, (h + 1) % 2
            rdmaT = pltpu.make_async_remote_copy(
                src_ref=commT.at[s], dst_ref=commT.at[r],
                send_sem=send_semsT.at[s], recv_sem=recv_semsT.at[r],
                device_id=(right,), device_id_type=pl.DeviceIdType.MESH)
            rdmaB = pltpu.make_async_remote_copy(
                src_ref=commB.at[s], dst_ref=commB.at[r],
                send_sem=send_semsB.at[s], recv_sem=recv_semsB.at[r],
                device_id=(left,), device_id_type=pl.DeviceIdType.MESH)
            rdmaT.start()
            rdmaB.start()
            rdmaT.wait()
            rdmaB.wait()
            gemm(commT, s, (my - h + N_DEV) % N_DEV, 0)
            gemm(commB, s, (my + h) % N_DEV, half)

        last = (N_DEV - 1) % 2
        gemm(commT, last, (my - (N_DEV - 1) + N_DEV) % N_DEV, 0)
        gemm(commB, last, (my + (N_DEV - 1)) % N_DEV, half)

    return pl.pallas_call(
        body,
        out_shape=jax.ShapeDtypeStruct((N_DEV * m_per, n_per), jnp.float32),
        in_specs=[
            pl.BlockSpec(memory_space=pltpu.ANY),
            pl.BlockSpec(memory_space=pltpu.VMEM),
        ],
        out_specs=pl.BlockSpec(memory_space=pltpu.VMEM),
        scratch_shapes=[
            pltpu.VMEM((2, half, k), x.dtype),
            pltpu.VMEM((2, half, k), x.dtype),
            pltpu.SemaphoreType.DMA((2,)),
            pltpu.SemaphoreType.DMA((2,)),
            pltpu.SemaphoreType.DMA((2,)),
            pltpu.SemaphoreType.DMA((2,)),
            pltpu.SemaphoreType.DMA((2,)),
        ],
        compiler_params=pltpu.CompilerParams(collective_id=0),
    )(x, w_mat)


# baseline (device time: 759667 ns/iter reference)
import jax
import jax.numpy as jnp
from jax import lax
from jax.experimental import pallas as pl
from jax.experimental.pallas import tpu as pltpu

N_DEV = 4
P = 256
NACC = 4


def kernel(x, w_mat):
    x = x.astype(jnp.bfloat16)
    w_mat = w_mat.astype(jnp.bfloat16)
    m_per, k = x.shape
    _, n_per = w_mat.shape
    half = m_per // 2
    npiece = half // P
    nown = m_per // P
    nt = (N_DEV - 1) * npiece

    def body(x_ref, w_ref, out_ref, commR, commL, ownbuf, acc,
             sendR, recvR, sendL, recvL, ownsems, outsems):
        my = lax.axis_index("i")
        left = (my + N_DEV - 1) % N_DEV
        right = (my + 1) % N_DEV

        barrier = pltpu.get_barrier_semaphore()
        for nbr in (left, right):
            pl.semaphore_signal(
                barrier, inc=1,
                device_id=(nbr,), device_id_type=pl.DeviceIdType.MESH)
        pl.semaphore_wait(barrier, 2)

        own_copies = {}

        def start_own(j):
            c = pltpu.make_async_copy(
                x_ref.at[pl.ds(j * P, P)], ownbuf.at[j % 2],
                ownsems.at[j % 2])
            c.start()
            own_copies[j] = c

        start_own(0)
        start_own(1)

        state = {"g": 0}
        acc_pending = {}

        def emit(src, row):
            slot = state["g"] % NACC
            if slot in acc_pending:
                acc_pending[slot].wait()
            acc[slot] = jnp.dot(src, w_ref[...],
                                preferred_element_type=jnp.float32)
            cp = pltpu.make_async_copy(
                acc.at[slot], out_ref.at[pl.ds(row, P)], outsems.at[slot])
            cp.start()
            acc_pending[slot] = cp
            state["g"] += 1

        for t in range(nt):
            p, h = divmod(t, N_DEV - 1)
            if h == 0:
                srcR = x_ref.at[pl.ds(p * P, P)]
                srcL = x_ref.at[pl.ds(half + p * P, P)]
            else:
                srcR = commR.at[(t - 1) % 2]
                srcL = commL.at[(t - 1) % 2]
            rdmaR = pltpu.make_async_remote_copy(
                src_ref=srcR, dst_ref=commR.at[t % 2],
                send_sem=sendR.at[t % 2], recv_sem=recvR.at[t % 2],
                device_id=(right,), device_id_type=pl.DeviceIdType.MESH)
            rdmaL = pltpu.make_async_remote_copy(
                src_ref=srcL, dst_ref=commL.at[t % 2],
                send_sem=sendL.at[t % 2], recv_sem=recvL.at[t % 2],
                device_id=(left,), device_id_type=pl.DeviceIdType.MESH)
            rdmaR.start()
            rdmaL.start()
            if t < nown:
                own_copies[t].wait()
                emit(ownbuf[t % 2], my * m_per + t * P)
                if t + 2 < nown:
                    start_own(t + 2)
            rdmaR.wait()
            rdmaL.wait()
            oR = (my + N_DEV - 1 - h) % N_DEV
            emit(commR[t % 2], oR * m_per + p * P)
            oL = (my + 1 + h) % N_DEV
            emit(commL[t % 2], oL * m_per + half + p * P)

        for cp in acc_pending.values():
            cp.wait()

    return pl.pallas_call(
        body,
        out_shape=jax.ShapeDtypeStruct((N_DEV * m_per, n_per), jnp.float32),
        in_specs=[
            pl.BlockSpec(memory_space=pl.ANY),
            pl.BlockSpec(memory_space=pltpu.MemorySpace.VMEM),
        ],
        out_specs=pl.BlockSpec(memory_space=pl.ANY),
        scratch_shapes=[
            pltpu.VMEM((2, P, k), jnp.bfloat16),
            pltpu.VMEM((2, P, k), jnp.bfloat16),
            pltpu.VMEM((2, P, k), jnp.bfloat16),
            pltpu.VMEM((NACC, P, n_per), jnp.float32),
            pltpu.SemaphoreType.DMA((2,)),
            pltpu.SemaphoreType.DMA((2,)),
            pltpu.SemaphoreType.DMA((2,)),
            pltpu.SemaphoreType.DMA((2,)),
            pltpu.SemaphoreType.DMA((2,)),
            pltpu.SemaphoreType.DMA((NACC,)),
        ],
        compiler_params=pltpu.CompilerParams(collective_id=0),
    )(x, w_mat)


# device time: 644941 ns/iter; 1.1779x vs baseline; 1.1779x over previous
import jax
import jax.numpy as jnp
from jax import lax
from jax.experimental import pallas as pl
from jax.experimental.pallas import tpu as pltpu

N_DEV = 4
P = 256
NACC = 4


def kernel(x, w_mat):
    x = x.astype(jnp.bfloat16)
    w_mat = w_mat.astype(jnp.bfloat16)
    m_per, k = x.shape
    _, n_per = w_mat.shape
    half = m_per // 2
    npiece = half // P
    nown = m_per // P
    nt = (N_DEV - 1) * npiece

    def body(x_ref, w_ref, out_ref, commR, commL, ownbuf, acc,
             sendR, recvR, sendL, recvL, ownsems, outsems):
        my = lax.axis_index("i")
        left = (my + N_DEV - 1) % N_DEV
        right = (my + 1) % N_DEV

        barrier = pltpu.get_barrier_semaphore()
        for nbr in (left, right):
            pl.semaphore_signal(
                barrier, inc=1,
                device_id=(nbr,), device_id_type=pl.DeviceIdType.MESH)
        pl.semaphore_wait(barrier, 2)

        own_copies = {}

        def start_own(j):
            c = pltpu.make_async_copy(
                x_ref.at[pl.ds(j * P, P)], ownbuf.at[j % 2],
                ownsems.at[j % 2])
            c.start()
            own_copies[j] = c

        start_own(0)
        start_own(1)

        state = {"g": 0}
        acc_pending = {}

        def emit(src, row):
            slot = state["g"] % NACC
            if slot in acc_pending:
                acc_pending[slot].wait()
            acc[slot] = jnp.dot(src, w_ref[...],
                                preferred_element_type=jnp.float32)
            cp = pltpu.make_async_copy(
                acc.at[slot], out_ref.at[pl.ds(row, P)], outsems.at[slot])
            cp.start()
            acc_pending[slot] = cp
            state["g"] += 1

        def start_transfer(t):
            p, h = divmod(t, N_DEV - 1)
            if h == 0:
                srcR = x_ref.at[pl.ds(p * P, P)]
                srcL = x_ref.at[pl.ds(half + p * P, P)]
            else:
                srcR = commR.at[(t - 1) % 2]
                srcL = commL.at[(t - 1) % 2]
            rdmaR = pltpu.make_async_remote_copy(
                src_ref=srcR, dst_ref=commR.at[t % 2],
                send_sem=sendR.at[t % 2], recv_sem=recvR.at[t % 2],
                device_id=(right,), device_id_type=pl.DeviceIdType.MESH)
            rdmaL = pltpu.make_async_remote_copy(
                src_ref=srcL, dst_ref=commL.at[t % 2],
                send_sem=sendL.at[t % 2], recv_sem=recvL.at[t % 2],
                device_id=(left,), device_id_type=pl.DeviceIdType.MESH)
            rdmaR.start()
            rdmaL.start()
            return rdmaR, rdmaL

        def gemm_received(t):
            p, h = divmod(t, N_DEV - 1)
            oR = (my + N_DEV - 1 - h) % N_DEV
            emit(commR[t % 2], oR * m_per + p * P)
            oL = (my + 1 + h) % N_DEV
            emit(commL[t % 2], oL * m_per + half + p * P)

        prevR, prevL = start_transfer(0)
        for t in range(1, nt):
            j = t - 1
            if j < nown:
                own_copies[j].wait()
                emit(ownbuf[j % 2], my * m_per + j * P)
                if j + 2 < nown:
                    start_own(j + 2)
            prevR.wait()
            prevL.wait()
            prevR, prevL = start_transfer(t)
            gemm_received(t - 1)
        prevR.wait()
        prevL.wait()
        gemm_received(nt - 1)

        for cp in acc_pending.values():
            cp.wait()

    return pl.pallas_call(
        body,
        out_shape=jax.ShapeDtypeStruct((N_DEV * m_per, n_per), jnp.float32),
        in_specs=[
            pl.BlockSpec(memory_space=pl.ANY),
            pl.BlockSpec(memory_space=pltpu.MemorySpace.VMEM),
        ],
        out_specs=pl.BlockSpec(memory_space=pl.ANY),
        scratch_shapes=[
            pltpu.VMEM((2, P, k), jnp.bfloat16),
            pltpu.VMEM((2, P, k), jnp.bfloat16),
            pltpu.VMEM((2, P, k), jnp.bfloat16),
            pltpu.VMEM((NACC, P, n_per), jnp.float32),
            pltpu.SemaphoreType.DMA((2,)),
            pltpu.SemaphoreType.DMA((2,)),
            pltpu.SemaphoreType.DMA((2,)),
            pltpu.SemaphoreType.DMA((2,)),
            pltpu.SemaphoreType.DMA((2,)),
            pltpu.SemaphoreType.DMA((NACC,)),
        ],
        compiler_params=pltpu.CompilerParams(collective_id=0),
    )(x, w_mat)
